# baseline (speedup 1.0000x reference)
import functools

import jax
import jax.numpy as jnp
from jax import lax
from jax.experimental import pallas as pl
from jax.experimental.pallas import tpu as pltpu
from jax.experimental.pallas import tpu_sc as plsc

B = 16384
HLEN = 200
ITEMS = 1000
VPAD = 1024
CATES = 10
CPAD = 16
EMBED = 64
EPS = 1e-3

NW = 32
LANES = 16
HALF = B // 2
ROWS_PER_W = HALF // NW
CHUNK = 16
NCHUNK = ROWS_PER_W // CHUNK

BT = 256
NBT = HALF // BT


NHB = 8
NCB = 2
TUNROLL = 8


def _sc_hist_body(hist_hbm, sl_hbm, out_hbm, *scr):
    hbufs = list(scr[0:NHB])
    sbufs = list(scr[NHB:2 * NHB])
    cbufs = list(scr[2 * NHB:2 * NHB + NCB])
    hsems = list(scr[2 * NHB + NCB:2 * NHB + NCB + NHB])
    csems = list(scr[2 * NHB + NCB + NHB:])

    wid = lax.axis_index("s") * 2 + lax.axis_index("c")
    lane = lax.broadcasted_iota(jnp.int32, (LANES,), 0)
    zeros16 = jnp.zeros((LANES,), jnp.float32)
    row0 = wid * ROWS_PER_W

    def start_fetch(ci, b):
        cb = jnp.minimum(ci, NCHUNK - 1)
        base = row0 + cb * CHUNK
        pltpu.async_copy(hist_hbm.at[pl.ds(base, CHUNK), :], hbufs[b], hsems[b])
        pltpu.async_copy(sl_hbm.at[pl.ds(base, CHUNK)], sbufs[b], hsems[b])

    def wait_fetch(b):
        base = row0
        pltpu.make_async_copy(hist_hbm.at[pl.ds(base, CHUNK), :],
                              hbufs[b], hsems[b]).wait()
        pltpu.make_async_copy(sl_hbm.at[pl.ds(base, CHUNK)],
                              sbufs[b], hsems[b]).wait()

    def scan_pass(hb, sb, cb):
        hv = hbufs[hb]
        cv = cbufs[cb]
        sl_vec = sbufs[sb][...]
        vals = jnp.ones((LANES,), jnp.float32)

        def t_body(t0, _):
            ids = []
            masks = []
            for k in range(TUNROLL):
                t = t0 * TUNROLL + k
                tvec = jnp.full((LANES,), t, jnp.int32)
                ids.append(plsc.load_gather(hv, [lane, tvec]))
                masks.append(tvec < sl_vec)
            for k in range(TUNROLL):
                plsc.addupdate_scatter(cv, [lane, ids[k]], vals, mask=masks[k])
            return 0

        lax.fori_loop(0, HLEN // TUNROLL, t_body, 0)

    def out_dma_start(ci, cb):
        base = row0 + ci * CHUNK
        pltpu.async_copy(
            cbufs[cb], out_hbm.at[pl.ds(base, CHUNK), :], csems[cb])

    def out_dma_wait(cb):
        pltpu.make_async_copy(
            cbufs[cb], out_hbm.at[pl.ds(0, CHUNK), :], csems[cb]).wait()

    def zero_body(j, _):
        for k in range(LANES):
            for cb in range(NCB):
                cbufs[cb][k, pl.ds(j * LANES, LANES)] = zeros16
        return 0

    lax.fori_loop(0, VPAD // LANES, zero_body, 0)

    for b in range(NHB):
        start_fetch(jnp.int32(b), b)

    def group_body(g, _):
        for b in range(NHB):
            ci = g * NHB + b
            cb = b % NCB
            ob = (b - NCB) % NHB

            def reclaim():
                out_dma_wait(cb)
                start_fetch(ci + NHB - NCB, ob)

            if b < NCB:
                pl.when(g > 0)(reclaim)
            else:
                reclaim()

            wait_fetch(b)
            scan_pass(b, b, cb)
            out_dma_start(ci, cb)
        return 0

    lax.fori_loop(0, NCHUNK // NHB, group_body, 0)

    for cb in range(NCB):
        out_dma_wait(cb)
    for b in range(NHB - NCB):
        wait_fetch(b)


def _sc_hist(hist_flat, sl_flat):
    mesh = plsc.VectorSubcoreMesh(core_axis_name="c", subcore_axis_name="s")
    scratch = (
        [pltpu.VMEM((CHUNK, HLEN), jnp.int32) for _ in range(NHB)]
        + [pltpu.VMEM((CHUNK,), jnp.int32) for _ in range(NHB)]
        + [pltpu.VMEM((CHUNK, VPAD), jnp.float32) for _ in range(NCB)]
        + [pltpu.SemaphoreType.DMA for _ in range(NHB + NCB)]
    )
    return pl.kernel(
        _sc_hist_body,
        out_type=jax.ShapeDtypeStruct((HALF, VPAD), jnp.float32),
        mesh=mesh,
        scratch_types=scratch,
        compiler_params=pltpu.CompilerParams(needs_layout_passes=False),
    )(hist_flat, sl_flat)


def _tc_body(counts_ref, item_ref, itab_ref, ctab_ref, clist_ref,
             ww_ref, bw_ref, w1_ref, b1_ref, w2_ref, b2_ref, w3_ref, b3_ref,
             out_ref):
    s_bn = 1.0 / jnp.sqrt(1.0 + EPS)
    s_pool = s_bn / HLEN

    cl = clist_ref[0, :]
    oh_c = (cl[:, None] == lax.broadcasted_iota(jnp.int32, (1, CPAD), 1)
            ).astype(jnp.float32)
    t_cate = jnp.dot(oh_c, ctab_ref[...],
                     preferred_element_type=jnp.float32)
    table = jnp.concatenate([itab_ref[...], t_cate], axis=1)

    snap = counts_ref[...]
    shift = jnp.concatenate(
        [jnp.zeros((NCB * CHUNK, VPAD), jnp.float32),
         snap[:BT - NCB * CHUNK, :]], axis=0)
    counts = snap - shift
    ue = jnp.dot(counts, table,
                 preferred_element_type=jnp.float32) * s_pool

    itm = item_ref[0, 0, :]
    oh_i = (itm[:, None] == lax.broadcasted_iota(jnp.int32, (1, VPAD), 1)
            ).astype(jnp.float32)
    ie = jnp.dot(oh_i, table, preferred_element_type=jnp.float32)

    ww = ww_ref[...]
    wide = (ue[:, 0:1] * ie[:, 0:1] * ww[0, 0]
            + ue[:, 127:128] * ie[:, 127:128] * ww[1, 0]
            + ue[:, 64:65] * ie[:, 64:65] * ww[2, 0]
            + bw_ref[0, 0])

    x = jnp.concatenate([ue, ie], axis=1) * s_bn
    h = jnp.maximum(jnp.dot(x, w1_ref[...],
                            preferred_element_type=jnp.float32)
                    + b1_ref[0, :], 0.0)
    h = jnp.maximum(jnp.dot(h, w2_ref[...],
                            preferred_element_type=jnp.float32)
                    + b2_ref[0, :], 0.0)
    d = jnp.dot(h, w3_ref[...], preferred_element_type=jnp.float32) + b3_ref[0, :]

    out_ref[...] = jax.nn.sigmoid(0.5 * (wide + d))


def _tc_net(counts, item_r, itab_pad, ctab_pad, clist_pad,
            Ww, bw, W1, b1, W2, b2, W3, b3):
    rep = lambda *shape: pl.BlockSpec(shape, lambda i: (0,) * len(shape))
    return pl.pallas_call(
        _tc_body,
        grid=(NBT,),
        in_specs=[
            pl.BlockSpec((BT, VPAD), lambda i: (i, 0)),
            pl.BlockSpec((1, 1, BT), lambda i: (i, 0, 0)),
            rep(VPAD, EMBED),
            rep(CPAD, EMBED),
            rep(1, VPAD),
            rep(3, 1),
            rep(1, 1),
            rep(2 * 2 * EMBED, 80),
            rep(1, 80),
            rep(80, 40),
            rep(1, 40),
            rep(40, 1),
            rep(1, 1),
        ],
        out_specs=pl.BlockSpec((BT, 1), lambda i: (i, 0)),
        out_shape=jax.ShapeDtypeStruct((HALF, 1), jnp.float32),
    )(counts, item_r, itab_pad, ctab_pad, clist_pad,
      Ww, bw, W1, b1, W2, b2, W3, b3)


def kernel(item_table, cate_table, Ww, bw, W1, b1, W2, b2, W3, b3,
           user, item, hist, sl, cate_list):
    del user

    itab_pad = jnp.pad(item_table, ((0, VPAD - ITEMS), (0, 0)))
    ctab_pad = jnp.pad(cate_table, ((0, CPAD - CATES), (0, 0)))
    clist_pad = jnp.pad(cate_list, (0, VPAD - ITEMS),
                        constant_values=CPAD + 1).reshape(1, VPAD)
    bw_r = bw.reshape(1, 1)
    b1_r = b1.reshape(1, -1)
    b2_r = b2.reshape(1, -1)
    b3_r = b3.reshape(1, 1)

    sl_flat = sl.reshape(-1)

    outs = []
    counts = [_sc_hist(hist[h * HALF:(h + 1) * HALF],
                       sl_flat[h * HALF:(h + 1) * HALF]) for h in range(2)]
    for h in range(2):
        item_r = item[h * HALF:(h + 1) * HALF].reshape(NBT, 1, BT)
        outs.append(_tc_net(counts[h], item_r, itab_pad, ctab_pad,
                            clist_pad, Ww, bw_r, W1, b1_r, W2, b2_r,
                            W3, b3_r))
    return jnp.concatenate(outs, axis=0)


# speedup vs baseline: 1.1323x; 1.1323x over previous
import functools

import jax
import jax.numpy as jnp
from jax import lax
from jax.experimental import pallas as pl
from jax.experimental.pallas import tpu as pltpu
from jax.experimental.pallas import tpu_sc as plsc

B = 16384
HLEN = 200
ITEMS = 1000
VPAD = 1024
CATES = 10
CPAD = 16
EMBED = 64
EPS = 1e-3

NW = 32
LANES = 16
ROWS_PER_W = B // NW
CHUNK = 16
NCHUNK = ROWS_PER_W // CHUNK

BT = 512
NBT = B // BT


NHB = 8
NCB = 2
TUNROLL = 10


def _sc_hist_body(hist_hbm, sl_hbm, out_hbm, *scr):
    hbufs = list(scr[0:NHB])
    sbufs = list(scr[NHB:2 * NHB])
    cbufs = list(scr[2 * NHB:2 * NHB + NCB])
    hsems = list(scr[2 * NHB + NCB:2 * NHB + NCB + NHB])
    csems = list(scr[2 * NHB + NCB + NHB:])

    wid = lax.axis_index("s") * 2 + lax.axis_index("c")
    lane = lax.broadcasted_iota(jnp.int32, (LANES,), 0)
    zeros16 = jnp.zeros((LANES,), jnp.float32)
    row0 = wid * ROWS_PER_W

    def start_fetch(ci, b):
        cb = jnp.minimum(ci, NCHUNK - 1)
        base = row0 + cb * CHUNK
        pltpu.async_copy(hist_hbm.at[pl.ds(base, CHUNK), :], hbufs[b], hsems[b])
        pltpu.async_copy(sl_hbm.at[pl.ds(base, CHUNK)], sbufs[b], hsems[b])

    def wait_fetch(b):
        base = row0
        pltpu.make_async_copy(hist_hbm.at[pl.ds(base, CHUNK), :],
                              hbufs[b], hsems[b]).wait()
        pltpu.make_async_copy(sl_hbm.at[pl.ds(base, CHUNK)],
                              sbufs[b], hsems[b]).wait()

    def scan_pass(hb, sb, cb):
        hv = hbufs[hb]
        cv = cbufs[cb]
        sl_vec = sbufs[sb][...]
        vals = jnp.ones((LANES,), jnp.float32)

        def t_body(t0, _):
            ids = []
            masks = []
            for k in range(TUNROLL):
                t = t0 * TUNROLL + k
                tvec = jnp.full((LANES,), t, jnp.int32)
                ids.append(plsc.load_gather(hv, [lane, tvec]))
                masks.append(tvec < sl_vec)
            for k in range(TUNROLL):
                plsc.addupdate_scatter(cv, [lane, ids[k]], vals, mask=masks[k])
            return 0

        lax.fori_loop(0, HLEN // TUNROLL, t_body, 0)

    def out_dma_start(ci, cb):
        base = row0 + ci * CHUNK
        pltpu.async_copy(
            cbufs[cb], out_hbm.at[pl.ds(base, CHUNK), :], csems[cb])

    def out_dma_wait(cb):
        pltpu.make_async_copy(
            cbufs[cb], out_hbm.at[pl.ds(0, CHUNK), :], csems[cb]).wait()

    def zero_body(j, _):
        for k in range(LANES):
            for cb in range(NCB):
                cbufs[cb][k, pl.ds(j * LANES, LANES)] = zeros16
        return 0

    lax.fori_loop(0, VPAD // LANES, zero_body, 0)

    for b in range(NHB):
        start_fetch(jnp.int32(b), b)

    def group_body(g, _):
        for b in range(NHB):
            ci = g * NHB + b
            cb = b % NCB
            ob = (b - NCB) % NHB

            def reclaim():
                out_dma_wait(cb)
                start_fetch(ci + NHB - NCB, ob)

            if b < NCB:
                pl.when(g > 0)(reclaim)
            else:
                reclaim()

            wait_fetch(b)
            scan_pass(b, b, cb)
            out_dma_start(ci, cb)
        return 0

    lax.fori_loop(0, NCHUNK // NHB, group_body, 0)

    for cb in range(NCB):
        out_dma_wait(cb)
    for b in range(NHB - NCB):
        wait_fetch(b)


def _sc_hist(hist_flat, sl_flat):
    mesh = plsc.VectorSubcoreMesh(core_axis_name="c", subcore_axis_name="s")
    scratch = (
        [pltpu.VMEM((CHUNK, HLEN), jnp.int32) for _ in range(NHB)]
        + [pltpu.VMEM((CHUNK,), jnp.int32) for _ in range(NHB)]
        + [pltpu.VMEM((CHUNK, VPAD), jnp.float32) for _ in range(NCB)]
        + [pltpu.SemaphoreType.DMA for _ in range(NHB + NCB)]
    )
    return pl.kernel(
        _sc_hist_body,
        out_type=jax.ShapeDtypeStruct((B, VPAD), jnp.float32),
        mesh=mesh,
        scratch_types=scratch,
        compiler_params=pltpu.CompilerParams(needs_layout_passes=False),
    )(hist_flat, sl_flat)


def _tc_body(counts_ref, item_ref, itab_ref, ctab_ref, clist_ref,
             ww_ref, bw_ref, w1_ref, b1_ref, w2_ref, b2_ref, w3_ref, b3_ref,
             out_ref):
    s_bn = 1.0 / jnp.sqrt(1.0 + EPS)
    s_pool = s_bn / HLEN

    cl = clist_ref[0, :]
    oh_c = (cl[:, None] == lax.broadcasted_iota(jnp.int32, (1, CPAD), 1)
            ).astype(jnp.float32)
    t_cate = jnp.dot(oh_c, ctab_ref[...],
                     preferred_element_type=jnp.float32)
    table = jnp.concatenate([itab_ref[...], t_cate], axis=1)

    snap = counts_ref[...]
    shift = jnp.concatenate(
        [jnp.zeros((NCB * CHUNK, VPAD), jnp.float32),
         snap[:BT - NCB * CHUNK, :]], axis=0)
    counts = snap - shift
    ue = jnp.dot(counts, table,
                 preferred_element_type=jnp.float32) * s_pool

    itm = item_ref[0, 0, :]
    oh_i = (itm[:, None] == lax.broadcasted_iota(jnp.int32, (1, VPAD), 1)
            ).astype(jnp.float32)
    ie = jnp.dot(oh_i, table, preferred_element_type=jnp.float32)

    ww = ww_ref[...]
    wide = (ue[:, 0:1] * ie[:, 0:1] * ww[0, 0]
            + ue[:, 127:128] * ie[:, 127:128] * ww[1, 0]
            + ue[:, 64:65] * ie[:, 64:65] * ww[2, 0]
            + bw_ref[0, 0])

    x = jnp.concatenate([ue, ie], axis=1) * s_bn
    h = jnp.maximum(jnp.dot(x, w1_ref[...],
                            preferred_element_type=jnp.float32)
                    + b1_ref[0, :], 0.0)
    h = jnp.maximum(jnp.dot(h, w2_ref[...],
                            preferred_element_type=jnp.float32)
                    + b2_ref[0, :], 0.0)
    d = jnp.dot(h, w3_ref[...], preferred_element_type=jnp.float32) + b3_ref[0, :]

    out_ref[...] = jax.nn.sigmoid(0.5 * (wide + d))


def _tc_net(counts, item_r, itab_pad, ctab_pad, clist_pad,
            Ww, bw, W1, b1, W2, b2, W3, b3):
    rep = lambda *shape: pl.BlockSpec(shape, lambda i: (0,) * len(shape))
    return pl.pallas_call(
        _tc_body,
        grid=(NBT,),
        in_specs=[
            pl.BlockSpec((BT, VPAD), lambda i: (i, 0)),
            pl.BlockSpec((1, 1, BT), lambda i: (i, 0, 0)),
            rep(VPAD, EMBED),
            rep(CPAD, EMBED),
            rep(1, VPAD),
            rep(3, 1),
            rep(1, 1),
            rep(2 * 2 * EMBED, 80),
            rep(1, 80),
            rep(80, 40),
            rep(1, 40),
            rep(40, 1),
            rep(1, 1),
        ],
        out_specs=pl.BlockSpec((BT, 1), lambda i: (i, 0)),
        out_shape=jax.ShapeDtypeStruct((B, 1), jnp.float32),
    )(counts, item_r, itab_pad, ctab_pad, clist_pad,
      Ww, bw, W1, b1, W2, b2, W3, b3)


def kernel(item_table, cate_table, Ww, bw, W1, b1, W2, b2, W3, b3,
           user, item, hist, sl, cate_list):
    del user

    counts = _sc_hist(hist, sl.reshape(-1))

    itab_pad = jnp.pad(item_table, ((0, VPAD - ITEMS), (0, 0)))
    ctab_pad = jnp.pad(cate_table, ((0, CPAD - CATES), (0, 0)))
    clist_pad = jnp.pad(cate_list, (0, VPAD - ITEMS),
                        constant_values=CPAD + 1).reshape(1, VPAD)
    item_r = item.reshape(NBT, 1, BT)
    bw_r = bw.reshape(1, 1)
    b1_r = b1.reshape(1, -1)
    b2_r = b2.reshape(1, -1)
    b3_r = b3.reshape(1, 1)

    return _tc_net(counts, item_r, itab_pad, ctab_pad, clist_pad,
                   Ww, bw_r, W1, b1_r, W2, b2_r, W3, b3_r)
